# Initial kernel scaffold; baseline (speedup 1.0000x reference)
import jax
import jax.numpy as jnp
from jax.experimental import pallas as pl


def kernel(input_vector, connections):
    raise NotImplementedError("write your pallas kernel here")


# speedup vs baseline: 3.5770x; 3.5770x over previous
import jax
import jax.numpy as jnp
from jax.experimental import pallas as pl
from jax.experimental.pallas import tpu as pltpu

N_TOKENS = 4096
INPUT_SIZE = 8192
NUM_COLS = 2048
K_ACTIVE = 40

ROWS_BLK = 256
J_BLK = 2048
J_STEPS = INPUT_SIZE // J_BLK
MAX_FINITE_BITS = 0x7F7FFFFF


def _body(x_ref, w_ref, o_ref, acc_ref):
    j = pl.program_id(1)

    @pl.when(j == 0)
    def _zero():
        acc_ref[...] = jnp.zeros_like(acc_ref)

    acc_ref[...] += jax.lax.dot_general(
        x_ref[...], w_ref[...],
        dimension_numbers=(((1,), (1,)), ((), ())),
        preferred_element_type=jnp.float32,
    )

    @pl.when(j == J_STEPS - 1)
    def _finish():
        acc = acc_ref[...]
        rows = acc.shape[0]
        lo0 = jnp.zeros((rows, 1), jnp.int32)
        hi0 = jnp.full((rows, 1), MAX_FINITE_BITS, dtype=jnp.int32)

        def step(_, lohi):
            lo, hi = lohi
            mid = lo + (hi - lo + 1) // 2
            t = jax.lax.bitcast_convert_type(mid, jnp.float32)
            cnt = jnp.sum((acc >= t).astype(jnp.int32), axis=1, keepdims=True)
            ge = cnt >= K_ACTIVE
            lo = jnp.where(ge, mid, lo)
            hi = jnp.where(ge, hi, mid - 1)
            return lo, hi

        lo, _ = jax.lax.fori_loop(0, 31, step, (lo0, hi0))
        thr = jax.lax.bitcast_convert_type(lo, jnp.float32)
        o_ref[...] = (acc >= thr).astype(jnp.int8)


def _pallas_kwta(input_vector, connections):
    return pl.pallas_call(
        _body,
        grid=(N_TOKENS // ROWS_BLK, J_STEPS),
        in_specs=[
            pl.BlockSpec((ROWS_BLK, J_BLK), lambda i, j: (i, j)),
            pl.BlockSpec((NUM_COLS, J_BLK), lambda i, j: (0, j)),
        ],
        out_specs=pl.BlockSpec((ROWS_BLK, NUM_COLS), lambda i, j: (i, 0)),
        out_shape=jax.ShapeDtypeStruct((N_TOKENS, NUM_COLS), jnp.int8),
        scratch_shapes=[pltpu.VMEM((ROWS_BLK, NUM_COLS), jnp.float32)],
    )(input_vector, connections)


def kernel(input_vector, connections):
    return _pallas_kwta(input_vector, connections).astype(jnp.bool_)


# speedup vs baseline: 5.0038x; 1.3989x over previous
import jax
import jax.numpy as jnp
from jax.experimental import pallas as pl
from jax.experimental.pallas import tpu as pltpu

N_TOKENS = 4096
INPUT_SIZE = 8192
NUM_COLS = 2048
K_ACTIVE = 40

ROWS_BLK = 512
J_BLK = 2048
J_STEPS = INPUT_SIZE // J_BLK
MAX_FINITE_BITS = 0x7F7FFFFF


def _body(x_ref, w_ref, o_ref, acc_ref):
    j = pl.program_id(1)

    @pl.when(j == 0)
    def _zero():
        acc_ref[...] = jnp.zeros_like(acc_ref)

    acc_ref[...] += jax.lax.dot_general(
        x_ref[...], w_ref[...],
        dimension_numbers=(((1,), (1,)), ((), ())),
        preferred_element_type=jnp.float32,
    )

    @pl.when(j == J_STEPS - 1)
    def _finish():
        acc = acc_ref[...]
        rows = acc.shape[0]
        lo0 = jnp.zeros((rows, 1), jnp.int32)
        hi0 = jnp.full((rows, 1), MAX_FINITE_BITS, dtype=jnp.int32)

        def step(_, lohi):
            lo, hi = lohi
            mid = lo + (hi - lo + 1) // 2
            t = jax.lax.bitcast_convert_type(mid, jnp.float32)
            cnt = jnp.sum((acc >= t).astype(jnp.int32), axis=1, keepdims=True)
            ge = cnt >= K_ACTIVE
            lo = jnp.where(ge, mid, lo)
            hi = jnp.where(ge, hi, mid - 1)
            return lo, hi

        lo, _ = jax.lax.fori_loop(0, 31, step, (lo0, hi0))
        thr = jax.lax.bitcast_convert_type(lo, jnp.float32)
        o_ref[...] = (acc >= thr).astype(jnp.int8)


def _pallas_kwta(input_vector, connections):
    return pl.pallas_call(
        _body,
        grid=(N_TOKENS // ROWS_BLK, J_STEPS),
        in_specs=[
            pl.BlockSpec((ROWS_BLK, J_BLK), lambda i, j: (i, j)),
            pl.BlockSpec((NUM_COLS, J_BLK), lambda i, j: (0, j)),
        ],
        out_specs=pl.BlockSpec((ROWS_BLK, NUM_COLS), lambda i, j: (i, 0)),
        out_shape=jax.ShapeDtypeStruct((N_TOKENS, NUM_COLS), jnp.int8),
        scratch_shapes=[pltpu.VMEM((ROWS_BLK, NUM_COLS), jnp.float32)],
    )(input_vector, connections)


def kernel(input_vector, connections):
    return _pallas_kwta(input_vector, connections).astype(jnp.bool_)


# speedup vs baseline: 5.7819x; 1.1555x over previous
import jax
import jax.numpy as jnp
from jax.experimental import pallas as pl
from jax.experimental.pallas import tpu as pltpu

N_TOKENS = 4096
INPUT_SIZE = 8192
NUM_COLS = 2048
K_ACTIVE = 40

ROWS_BLK = 1024
J_BLK = 1024
J_STEPS = INPUT_SIZE // J_BLK
MAX_FINITE_BITS = 0x7F7FFFFF


def _body(x_ref, w_ref, o_ref, acc_ref):
    j = pl.program_id(1)

    @pl.when(j == 0)
    def _zero():
        acc_ref[...] = jnp.zeros_like(acc_ref)

    acc_ref[...] += jax.lax.dot_general(
        x_ref[...], w_ref[...],
        dimension_numbers=(((1,), (1,)), ((), ())),
        preferred_element_type=jnp.float32,
    )

    @pl.when(j == J_STEPS - 1)
    def _finish():
        acc = acc_ref[...]
        rows = acc.shape[0]
        lo0 = jnp.zeros((rows, 1), jnp.int32)
        hi0 = jnp.full((rows, 1), MAX_FINITE_BITS, dtype=jnp.int32)

        def step(_, lohi):
            lo, hi = lohi
            mid = lo + (hi - lo + 1) // 2
            t = jax.lax.bitcast_convert_type(mid, jnp.float32)
            cnt = jnp.sum((acc >= t).astype(jnp.int32), axis=1, keepdims=True)
            ge = cnt >= K_ACTIVE
            lo = jnp.where(ge, mid, lo)
            hi = jnp.where(ge, hi, mid - 1)
            return lo, hi

        lo, _ = jax.lax.fori_loop(0, 31, step, (lo0, hi0))
        thr = jax.lax.bitcast_convert_type(lo, jnp.float32)
        o_ref[...] = (acc >= thr).astype(jnp.int8)


def _pallas_kwta(input_vector, connections):
    return pl.pallas_call(
        _body,
        grid=(N_TOKENS // ROWS_BLK, J_STEPS),
        in_specs=[
            pl.BlockSpec((ROWS_BLK, J_BLK), lambda i, j: (i, j)),
            pl.BlockSpec((NUM_COLS, J_BLK), lambda i, j: (0, j)),
        ],
        out_specs=pl.BlockSpec((ROWS_BLK, NUM_COLS), lambda i, j: (i, 0)),
        out_shape=jax.ShapeDtypeStruct((N_TOKENS, NUM_COLS), jnp.int8),
        scratch_shapes=[pltpu.VMEM((ROWS_BLK, NUM_COLS), jnp.float32)],
    )(input_vector, connections)


def kernel(input_vector, connections):
    return _pallas_kwta(input_vector, connections).astype(jnp.bool_)
